# Initial kernel scaffold; baseline (speedup 1.0000x reference)
import jax
import jax.numpy as jnp
from jax.experimental import pallas as pl


def kernel(x, W_enc, b_enc, W_dec, b_dec, input_mean, input_std):
    raise NotImplementedError("write your pallas kernel here")


# speedup vs baseline: 8.7691x; 8.7691x over previous
import functools

import jax
import jax.numpy as jnp
from jax.experimental import pallas as pl
from jax.experimental.pallas import tpu as pltpu

K = 32


def _encode_body(x_ref, we_ref, be_ref, mean_ref, std_ref, bdec_ref, h_ref):
    xc = (x_ref[...] - mean_ref[...]) / std_ref[...] - bdec_ref[...]
    h = jax.lax.dot_general(
        xc, we_ref[...], (((1,), (1,)), ((), ())),
        preferred_element_type=jnp.float32)
    h_ref[...] = h + be_ref[...]


def _bitsearch_kth(m, k, axes):
    cnt0 = jnp.sum((m >= 0).astype(jnp.int32), axis=axes, keepdims=True)
    t0 = jnp.where(cnt0 >= k, jnp.int32(0), jnp.int32(-2147483648))

    def bit_body(b, t):
        cand = t + (jnp.int32(1) << (30 - b))
        cnt = jnp.sum((m >= cand).astype(jnp.int32), axis=axes, keepdims=True)
        return jnp.where(cnt >= k, cand, t)

    return jax.lax.fori_loop(0, 31, bit_body, t0)


def _select_decode_body(h_ref, wd_ref, bdec_ref, hs_ref, xhat_ref):
    h = h_ref[...]
    BR, N = h.shape
    i = jax.lax.bitcast_convert_type(h, jnp.int32)
    m = jnp.where(i >= 0, i, i ^ jnp.int32(0x7FFFFFFF))

    NL = 128
    d3 = m.reshape(BR, N // NL, NL)
    cands = []
    for _ in range(8):
        mx = jnp.max(d3, axis=1, keepdims=True)
        cands.append(mx)
        d3 = jnp.where(d3 == mx, jnp.int32(-2147483648), d3)
    C = jnp.concatenate(cands, axis=1)

    t_cand = _bitsearch_kth(C, K, (1, 2)).reshape(BR, 1)
    cnt_full = jnp.sum((m >= t_cand).astype(jnp.int32), axis=1, keepdims=True)
    ok = jnp.all(cnt_full >= K)

    t = jax.lax.cond(ok, lambda: t_cand,
                     lambda: _bitsearch_kth(m, K, (1,)))
    hs = jnp.where(m >= t, h, 0.0)
    hs_ref[...] = hs
    xhat = jax.lax.dot_general(
        hs, wd_ref[...], (((1,), (1,)), ((), ())),
        preferred_element_type=jnp.float32)
    xhat_ref[...] = xhat + bdec_ref[...]


def _topk_sae(x, W_enc, b_enc, W_dec, b_dec, input_mean, input_std,
              interpret=False):
    B, D = x.shape
    N = W_enc.shape[0]
    BN = 512
    BR = 64

    be2 = b_enc.reshape(1, N)
    bd2 = b_dec.reshape(1, D)
    mean2 = input_mean.reshape(1, D)
    std2 = input_std.reshape(1, D)

    h_pre = pl.pallas_call(
        _encode_body,
        grid=(N // BN,),
        in_specs=[
            pl.BlockSpec((B, D), lambda j: (0, 0)),
            pl.BlockSpec((BN, D), lambda j: (j, 0)),
            pl.BlockSpec((1, BN), lambda j: (0, j)),
            pl.BlockSpec((1, D), lambda j: (0, 0)),
            pl.BlockSpec((1, D), lambda j: (0, 0)),
            pl.BlockSpec((1, D), lambda j: (0, 0)),
        ],
        out_specs=pl.BlockSpec((B, BN), lambda j: (0, j)),
        out_shape=jax.ShapeDtypeStruct((B, N), jnp.float32),
        compiler_params=pltpu.CompilerParams(
            dimension_semantics=("parallel",)),
        interpret=interpret,
    )(x, W_enc, be2, mean2, std2, bd2)

    h_sparse, x_hat = pl.pallas_call(
        _select_decode_body,
        grid=(B // BR,),
        in_specs=[
            pl.BlockSpec((BR, N), lambda i: (i, 0)),
            pl.BlockSpec((D, N), lambda i: (0, 0)),
            pl.BlockSpec((1, D), lambda i: (0, 0)),
        ],
        out_specs=[
            pl.BlockSpec((BR, N), lambda i: (i, 0)),
            pl.BlockSpec((BR, D), lambda i: (i, 0)),
        ],
        out_shape=[
            jax.ShapeDtypeStruct((B, N), jnp.float32),
            jax.ShapeDtypeStruct((B, D), jnp.float32),
        ],
        compiler_params=pltpu.CompilerParams(
            dimension_semantics=("parallel",)),
        interpret=interpret,
    )(h_pre, W_dec, bd2)

    return (x_hat, h_sparse, h_pre)


def kernel(x, W_enc, b_enc, W_dec, b_dec, input_mean, input_std):
    return _topk_sae(x, W_enc, b_enc, W_dec, b_dec, input_mean, input_std)


# speedup vs baseline: 11.0889x; 1.2645x over previous
import functools

import jax
import jax.numpy as jnp
from jax.experimental import pallas as pl
from jax.experimental.pallas import tpu as pltpu

K = 32


def _encode_body(x_ref, we_ref, be_ref, mean_ref, std_ref, bdec_ref, h_ref):
    xc = (x_ref[...] - mean_ref[...]) / std_ref[...] - bdec_ref[...]
    h = jax.lax.dot_general(
        xc, we_ref[...], (((1,), (1,)), ((), ())),
        preferred_element_type=jnp.float32)
    h_ref[...] = h + be_ref[...]


def _bitsearch_kth(m, k, axes):
    cnt0 = jnp.sum((m >= 0).astype(jnp.int32), axis=axes, keepdims=True)
    t0 = jnp.where(cnt0 >= k, jnp.int32(0), jnp.int32(-2147483648))

    def bit_body(b, t):
        cand = t + (jnp.int32(1) << (30 - b))
        cnt = jnp.sum((m >= cand).astype(jnp.int32), axis=axes, keepdims=True)
        return jnp.where(cnt >= k, cand, t)

    return jax.lax.fori_loop(0, 31, bit_body, t0)


def _select_decode_body(h_ref, wd_ref, bdec_ref, hs_ref, xhat_ref):
    h = h_ref[...]
    BR, N = h.shape
    i = jax.lax.bitcast_convert_type(h, jnp.int32)
    m = jnp.where(i >= 0, i, i ^ jnp.int32(0x7FFFFFFF))

    NL = 128
    d3 = m.reshape(BR, N // NL, NL)
    cands = []
    for _ in range(8):
        mx = jnp.max(d3, axis=1, keepdims=True)
        cands.append(mx)
        d3 = jnp.where(d3 == mx, jnp.int32(-2147483648), d3)
    C = jnp.concatenate(cands, axis=1)

    Ct = jnp.transpose(C.reshape(BR, 8 * NL), (1, 0))
    t_cand = jnp.transpose(_bitsearch_kth(Ct, K, (0,)), (1, 0))
    cnt_full = jnp.sum((m >= t_cand).astype(jnp.int32), axis=1, keepdims=True)
    ok = jnp.all(cnt_full >= K)

    t = jax.lax.cond(ok, lambda: t_cand,
                     lambda: _bitsearch_kth(m, K, (1,)))
    hs = jnp.where(m >= t, h, 0.0)
    hs_ref[...] = hs
    xhat = jax.lax.dot_general(
        hs, wd_ref[...], (((1,), (1,)), ((), ())),
        preferred_element_type=jnp.float32)
    xhat_ref[...] = xhat + bdec_ref[...]


def _topk_sae(x, W_enc, b_enc, W_dec, b_dec, input_mean, input_std,
              interpret=False):
    B, D = x.shape
    N = W_enc.shape[0]
    BN = 512
    BR = 64

    be2 = b_enc.reshape(1, N)
    bd2 = b_dec.reshape(1, D)
    mean2 = input_mean.reshape(1, D)
    std2 = input_std.reshape(1, D)

    h_pre = pl.pallas_call(
        _encode_body,
        grid=(N // BN,),
        in_specs=[
            pl.BlockSpec((B, D), lambda j: (0, 0)),
            pl.BlockSpec((BN, D), lambda j: (j, 0)),
            pl.BlockSpec((1, BN), lambda j: (0, j)),
            pl.BlockSpec((1, D), lambda j: (0, 0)),
            pl.BlockSpec((1, D), lambda j: (0, 0)),
            pl.BlockSpec((1, D), lambda j: (0, 0)),
        ],
        out_specs=pl.BlockSpec((B, BN), lambda j: (0, j)),
        out_shape=jax.ShapeDtypeStruct((B, N), jnp.float32),
        compiler_params=pltpu.CompilerParams(
            dimension_semantics=("parallel",)),
        interpret=interpret,
    )(x, W_enc, be2, mean2, std2, bd2)

    h_sparse, x_hat = pl.pallas_call(
        _select_decode_body,
        grid=(B // BR,),
        in_specs=[
            pl.BlockSpec((BR, N), lambda i: (i, 0)),
            pl.BlockSpec((D, N), lambda i: (0, 0)),
            pl.BlockSpec((1, D), lambda i: (0, 0)),
        ],
        out_specs=[
            pl.BlockSpec((BR, N), lambda i: (i, 0)),
            pl.BlockSpec((BR, D), lambda i: (i, 0)),
        ],
        out_shape=[
            jax.ShapeDtypeStruct((B, N), jnp.float32),
            jax.ShapeDtypeStruct((B, D), jnp.float32),
        ],
        compiler_params=pltpu.CompilerParams(
            dimension_semantics=("parallel",)),
        interpret=interpret,
    )(h_pre, W_dec, bd2)

    return (x_hat, h_sparse, h_pre)


def kernel(x, W_enc, b_enc, W_dec, b_dec, input_mean, input_std):
    return _topk_sae(x, W_enc, b_enc, W_dec, b_dec, input_mean, input_std)
